# baseline (speedup 1.0000x reference)
import functools

import jax
import jax.numpy as jnp
from jax import lax
from jax.experimental import pallas as pl
from jax.experimental.pallas import tpu as pltpu
from jax.experimental.pallas import tpu_sc as plsc

N = 3200000
G = 100000
GPAD = 102400
ROWS = 25000
CW = 384
RB = 1000

NUM_SC = 2
TILES = 16
NUM_W = NUM_SC * TILES
PER_W = N // NUM_W
CH = 20000
SLICE = GPAD // TILES


def _dense_body(p_ref, charges_ref, vals_ref):
    blk = p_ref[...]
    k3 = lax.broadcasted_iota(jnp.int32, (CW, 128), 0) // 3
    j = lax.broadcasted_iota(jnp.int32, (CW, 128), 1)
    sel = (k3 == j).astype(jnp.float32)
    charges_ref[...] = jnp.dot(blk, sel, precision=lax.Precision.HIGHEST)
    vals = jnp.dot(blk * blk, sel, precision=lax.Precision.HIGHEST) * 0.25
    vals_ref[...] = vals.reshape(RB * 128)


_dense_call = pl.pallas_call(
    _dense_body,
    grid=(ROWS // RB,),
    in_specs=[pl.BlockSpec((RB, CW), lambda i: (i, 0))],
    out_specs=[
        pl.BlockSpec((RB, 128), lambda i: (i, 0)),
        pl.BlockSpec((RB * 128,), lambda i: (i,)),
    ],
    out_shape=[
        jax.ShapeDtypeStruct((ROWS, 128), jnp.float32),
        jax.ShapeDtypeStruct((N,), jnp.float32),
    ],
)


def _scatter_body(vals_hbm, batch_hbm, out_hbm, idx_v, val_v, buf_v, acc):
    cid = lax.axis_index("c")
    sid = lax.axis_index("s")
    wid = cid * TILES + sid

    def _zero(i, carry):
        buf_v[pl.ds(i * 16, 16)] = jnp.zeros((16,), jnp.float32)
        return carry

    lax.fori_loop(0, SLICE // 16, _zero, 0)
    pltpu.sync_copy(buf_v, acc.at[pl.ds(sid * SLICE, SLICE)])
    plsc.subcore_barrier()

    for k in range(PER_W // CH):
        base = wid * PER_W + k * CH
        pltpu.sync_copy(batch_hbm.at[pl.ds(base, CH)], idx_v)
        pltpu.sync_copy(vals_hbm.at[pl.ds(base, CH)], val_v)
        pltpu.sync_copy(val_v, acc.at[idx_v], add=True)
    plsc.subcore_barrier()

    pltpu.sync_copy(acc.at[pl.ds(sid * SLICE, SLICE)], buf_v)
    pltpu.sync_copy(buf_v, out_hbm.at[cid, pl.ds(sid * SLICE, SLICE)])


_scatter_call = pl.kernel(
    _scatter_body,
    out_type=jax.ShapeDtypeStruct((NUM_SC, GPAD), jnp.float32),
    mesh=plsc.VectorSubcoreMesh(core_axis_name="c", subcore_axis_name="s"),
    compiler_params=pltpu.CompilerParams(use_tc_tiling_on_sc=True),
    scratch_types=[
        pltpu.VMEM((CH,), jnp.int32),
        pltpu.VMEM((CH,), jnp.float32),
        pltpu.VMEM((SLICE,), jnp.float32),
        pltpu.VMEM_SHARED((GPAD,), jnp.float32),
    ],
)


def _combine_body(p_ref, out_ref):
    out_ref[...] = p_ref[0] + p_ref[1]


_combine_call = pl.pallas_call(
    _combine_body,
    in_specs=[pl.BlockSpec((NUM_SC, GPAD // 128, 128), lambda: (0, 0, 0))],
    out_specs=pl.BlockSpec((GPAD // 128, 128), lambda: (0, 0)),
    out_shape=jax.ShapeDtypeStruct((GPAD // 128, 128), jnp.float32),
)


def kernel(positions, atomic_numbers, batch):
    del atomic_numbers
    pos2 = positions.reshape(ROWS, CW)
    charges2, vals_flat = _dense_call(pos2)
    partials = _scatter_call(vals_flat, batch.astype(jnp.int32))
    combined = _combine_call(partials.reshape(NUM_SC, GPAD // 128, 128))
    energies = combined.reshape(GPAD)[:G].reshape(G, 1)
    node_charges = charges2.reshape(N, 1)
    return (energies, node_charges)


# speedup vs baseline: 53.5585x; 53.5585x over previous
import jax
import jax.numpy as jnp
from jax import lax
from jax.experimental import pallas as pl
from jax.experimental.pallas import tpu as pltpu
from jax.experimental.pallas import tpu_sc as plsc

N = 3200000
G = 100000
GPAD = 102400
BLK = 128000

NUM_SC = 2
TILES = 16
NUM_W = NUM_SC * TILES
PER_W = N // NUM_W
CH = 20000
SLICE = GPAD // TILES


def _dense_body(x_ref, y_ref, z_ref, charges_ref, vals_ref):
    x = x_ref[...]
    y = y_ref[...]
    z = z_ref[...]
    charges_ref[...] = x + y + z
    vals_ref[...] = (x * x + y * y + z * z) * 0.25


_dense_call = pl.pallas_call(
    _dense_body,
    grid=(N // BLK,),
    in_specs=[
        pl.BlockSpec((BLK,), lambda i: (i,)),
        pl.BlockSpec((BLK,), lambda i: (i,)),
        pl.BlockSpec((BLK,), lambda i: (i,)),
    ],
    out_specs=[
        pl.BlockSpec((BLK,), lambda i: (i,)),
        pl.BlockSpec((BLK,), lambda i: (i,)),
    ],
    out_shape=[
        jax.ShapeDtypeStruct((N,), jnp.float32),
        jax.ShapeDtypeStruct((N,), jnp.float32),
    ],
)


def _scatter_body(vals_hbm, batch_hbm, out_hbm, idx_v, val_v, buf_v, acc):
    cid = lax.axis_index("c")
    sid = lax.axis_index("s")
    wid = cid * TILES + sid

    def _zero(i, carry):
        buf_v[pl.ds(i * 16, 16)] = jnp.zeros((16,), jnp.float32)
        return carry

    lax.fori_loop(0, SLICE // 16, _zero, 0)
    pltpu.sync_copy(buf_v, acc.at[pl.ds(sid * SLICE, SLICE)])
    plsc.subcore_barrier()

    for k in range(PER_W // CH):
        base = wid * PER_W + k * CH
        pltpu.sync_copy(batch_hbm.at[pl.ds(base, CH)], idx_v)
        pltpu.sync_copy(vals_hbm.at[pl.ds(base, CH)], val_v)
        pltpu.sync_copy(val_v, acc.at[idx_v], add=True)
    plsc.subcore_barrier()

    pltpu.sync_copy(acc.at[pl.ds(sid * SLICE, SLICE)], buf_v)
    pltpu.sync_copy(buf_v, out_hbm.at[cid, pl.ds(sid * SLICE, SLICE)])


_scatter_call = pl.kernel(
    _scatter_body,
    out_type=jax.ShapeDtypeStruct((NUM_SC, GPAD), jnp.float32),
    mesh=plsc.VectorSubcoreMesh(core_axis_name="c", subcore_axis_name="s"),
    scratch_types=[
        pltpu.VMEM((CH,), jnp.int32),
        pltpu.VMEM((CH,), jnp.float32),
        pltpu.VMEM((SLICE,), jnp.float32),
        pltpu.VMEM_SHARED((GPAD,), jnp.float32),
    ],
)


def _combine_body(p_ref, out_ref):
    out_ref[...] = p_ref[0] + p_ref[1]


_combine_call = pl.pallas_call(
    _combine_body,
    in_specs=[pl.BlockSpec((NUM_SC, GPAD // 128, 128), lambda: (0, 0, 0))],
    out_specs=pl.BlockSpec((GPAD // 128, 128), lambda: (0, 0)),
    out_shape=jax.ShapeDtypeStruct((GPAD // 128, 128), jnp.float32),
)


def kernel(positions, atomic_numbers, batch):
    del atomic_numbers
    x = positions[:, 0]
    y = positions[:, 1]
    z = positions[:, 2]
    charges_flat, vals_flat = _dense_call(x, y, z)
    partials = _scatter_call(vals_flat, batch.astype(jnp.int32))
    combined = _combine_call(partials.reshape(NUM_SC, GPAD // 128, 128))
    energies = combined.reshape(GPAD)[:G].reshape(G, 1)
    node_charges = charges_flat.reshape(N, 1)
    return (energies, node_charges)


# speedup vs baseline: 57.3181x; 1.0702x over previous
import jax
import jax.numpy as jnp
from jax import lax
from jax.experimental import pallas as pl
from jax.experimental.pallas import tpu as pltpu
from jax.experimental.pallas import tpu_sc as plsc

N = 3200000
G = 100000
GPAD = 102400
BLK = 128000

NUM_SC = 2
TILES = 16
NUM_W = NUM_SC * TILES
PER_W = N // NUM_W
CH = 25000
NCH = PER_W // CH
SLICE = GPAD // TILES


def _dense_body(x_ref, y_ref, z_ref, charges_ref, vals_ref):
    x = x_ref[...]
    y = y_ref[...]
    z = z_ref[...]
    charges_ref[...] = x + y + z
    vals_ref[...] = (x * x + y * y + z * z) * 0.25


_dense_call = pl.pallas_call(
    _dense_body,
    grid=(N // BLK,),
    in_specs=[
        pl.BlockSpec((BLK,), lambda i: (i,)),
        pl.BlockSpec((BLK,), lambda i: (i,)),
        pl.BlockSpec((BLK,), lambda i: (i,)),
    ],
    out_specs=[
        pl.BlockSpec((BLK,), lambda i: (i,)),
        pl.BlockSpec((BLK,), lambda i: (i,)),
    ],
    out_shape=[
        jax.ShapeDtypeStruct((N,), jnp.float32),
        jax.ShapeDtypeStruct((N,), jnp.float32),
    ],
)


def _scatter_body(vals_hbm, batch_hbm, out_hbm, idx0_v, idx1_v, val0_v,
                  val1_v, buf_v, acc, sem_i, sem_v):
    cid = lax.axis_index("c")
    sid = lax.axis_index("s")
    wid = cid * TILES + sid
    idx_bufs = (idx0_v, idx1_v)
    val_bufs = (val0_v, val1_v)

    def _start_load(k):
        base = wid * PER_W + k * CH
        b = k % 2
        return (
            pltpu.async_copy(batch_hbm.at[pl.ds(base, CH)], idx_bufs[b],
                             sem_i.at[b]),
            pltpu.async_copy(vals_hbm.at[pl.ds(base, CH)], val_bufs[b],
                             sem_v.at[b]),
        )

    handles = {0: _start_load(0)}

    def _zero(i, carry):
        buf_v[pl.ds(i * 16, 16)] = jnp.zeros((16,), jnp.float32)
        return carry

    lax.fori_loop(0, SLICE // 16, _zero, 0)
    pltpu.sync_copy(buf_v, acc.at[pl.ds(sid * SLICE, SLICE)])
    plsc.subcore_barrier()

    for k in range(NCH):
        if k + 1 < NCH:
            handles[k + 1] = _start_load(k + 1)
        hi, hv = handles.pop(k)
        hi.wait()
        hv.wait()
        b = k % 2
        pltpu.sync_copy(val_bufs[b], acc.at[idx_bufs[b]], add=True)
    plsc.subcore_barrier()

    pltpu.sync_copy(acc.at[pl.ds(sid * SLICE, SLICE)], buf_v)
    pltpu.sync_copy(buf_v, out_hbm.at[cid, pl.ds(sid * SLICE, SLICE)])


_scatter_call = pl.kernel(
    _scatter_body,
    out_type=jax.ShapeDtypeStruct((NUM_SC, GPAD), jnp.float32),
    mesh=plsc.VectorSubcoreMesh(core_axis_name="c", subcore_axis_name="s"),
    scratch_types=[
        pltpu.VMEM((CH,), jnp.int32),
        pltpu.VMEM((CH,), jnp.int32),
        pltpu.VMEM((CH,), jnp.float32),
        pltpu.VMEM((CH,), jnp.float32),
        pltpu.VMEM((SLICE,), jnp.float32),
        pltpu.VMEM_SHARED((GPAD,), jnp.float32),
        pltpu.SemaphoreType.DMA((2,)),
        pltpu.SemaphoreType.DMA((2,)),
    ],
)


def _combine_body(p_ref, out_ref):
    out_ref[...] = p_ref[0] + p_ref[1]


_combine_call = pl.pallas_call(
    _combine_body,
    in_specs=[pl.BlockSpec((NUM_SC, GPAD // 128, 128), lambda: (0, 0, 0))],
    out_specs=pl.BlockSpec((GPAD // 128, 128), lambda: (0, 0)),
    out_shape=jax.ShapeDtypeStruct((GPAD // 128, 128), jnp.float32),
)


def kernel(positions, atomic_numbers, batch):
    del atomic_numbers
    x = positions[:, 0]
    y = positions[:, 1]
    z = positions[:, 2]
    charges_flat, vals_flat = _dense_call(x, y, z)
    partials = _scatter_call(vals_flat, batch.astype(jnp.int32))
    combined = _combine_call(partials.reshape(NUM_SC, GPAD // 128, 128))
    energies = combined.reshape(GPAD)[:G].reshape(G, 1)
    node_charges = charges_flat.reshape(N, 1)
    return (energies, node_charges)


# speedup vs baseline: 59.5028x; 1.0381x over previous
import jax
import jax.numpy as jnp
from jax import lax
from jax.experimental import pallas as pl
from jax.experimental.pallas import tpu as pltpu
from jax.experimental.pallas import tpu_sc as plsc

N = 3200000
G = 100000
GPAD = 102400
BLK = 640000

NUM_SC = 2
TILES = 16
NUM_W = NUM_SC * TILES
PER_W = N // NUM_W
CH = 25000
NCH = PER_W // CH
SLICE = GPAD // TILES


def _dense_body(x_ref, y_ref, z_ref, charges_ref, vals_ref):
    x = x_ref[...]
    y = y_ref[...]
    z = z_ref[...]
    charges_ref[...] = x + y + z
    vals_ref[...] = (x * x + y * y + z * z) * 0.25


_dense_call = pl.pallas_call(
    _dense_body,
    grid=(N // BLK,),
    in_specs=[
        pl.BlockSpec((BLK,), lambda i: (i,)),
        pl.BlockSpec((BLK,), lambda i: (i,)),
        pl.BlockSpec((BLK,), lambda i: (i,)),
    ],
    out_specs=[
        pl.BlockSpec((BLK,), lambda i: (i,)),
        pl.BlockSpec((BLK,), lambda i: (i,)),
    ],
    out_shape=[
        jax.ShapeDtypeStruct((N,), jnp.float32),
        jax.ShapeDtypeStruct((N,), jnp.float32),
    ],
)


def _scatter_body(vals_hbm, batch_hbm, out_hbm, idx0_v, idx1_v, val0_v,
                  val1_v, buf_v, acc, sem_i, sem_v):
    cid = lax.axis_index("c")
    sid = lax.axis_index("s")
    wid = cid * TILES + sid
    idx_bufs = (idx0_v, idx1_v)
    val_bufs = (val0_v, val1_v)

    def _start_load(k):
        base = wid * PER_W + k * CH
        b = k % 2
        return (
            pltpu.async_copy(batch_hbm.at[pl.ds(base, CH)], idx_bufs[b],
                             sem_i.at[b]),
            pltpu.async_copy(vals_hbm.at[pl.ds(base, CH)], val_bufs[b],
                             sem_v.at[b]),
        )

    handles = {0: _start_load(0)}

    def _zero(i, carry):
        buf_v[pl.ds(i * 16, 16)] = jnp.zeros((16,), jnp.float32)
        return carry

    lax.fori_loop(0, SLICE // 16, _zero, 0)
    pltpu.sync_copy(buf_v, acc.at[pl.ds(sid * SLICE, SLICE)])
    plsc.subcore_barrier()

    for k in range(NCH):
        if k + 1 < NCH:
            handles[k + 1] = _start_load(k + 1)
        hi, hv = handles.pop(k)
        hi.wait()
        hv.wait()
        b = k % 2
        pltpu.sync_copy(val_bufs[b], acc.at[idx_bufs[b]], add=True)
    plsc.subcore_barrier()

    pltpu.sync_copy(acc.at[pl.ds(sid * SLICE, SLICE)], buf_v)
    pltpu.sync_copy(buf_v, out_hbm.at[cid, pl.ds(sid * SLICE, SLICE)])


_scatter_call = pl.kernel(
    _scatter_body,
    out_type=jax.ShapeDtypeStruct((NUM_SC, GPAD), jnp.float32),
    mesh=plsc.VectorSubcoreMesh(core_axis_name="c", subcore_axis_name="s"),
    scratch_types=[
        pltpu.VMEM((CH,), jnp.int32),
        pltpu.VMEM((CH,), jnp.int32),
        pltpu.VMEM((CH,), jnp.float32),
        pltpu.VMEM((CH,), jnp.float32),
        pltpu.VMEM((SLICE,), jnp.float32),
        pltpu.VMEM_SHARED((GPAD,), jnp.float32),
        pltpu.SemaphoreType.DMA((2,)),
        pltpu.SemaphoreType.DMA((2,)),
    ],
)


def _combine_body(p_ref, out_ref):
    out_ref[...] = p_ref[0] + p_ref[1]


_combine_call = pl.pallas_call(
    _combine_body,
    in_specs=[pl.BlockSpec((NUM_SC, GPAD // 128, 128), lambda: (0, 0, 0))],
    out_specs=pl.BlockSpec((GPAD // 128, 128), lambda: (0, 0)),
    out_shape=jax.ShapeDtypeStruct((GPAD // 128, 128), jnp.float32),
)


def kernel(positions, atomic_numbers, batch):
    del atomic_numbers
    x = positions[:, 0]
    y = positions[:, 1]
    z = positions[:, 2]
    charges_flat, vals_flat = _dense_call(x, y, z)
    partials = _scatter_call(vals_flat, batch.astype(jnp.int32))
    combined = _combine_call(partials.reshape(NUM_SC, GPAD // 128, 128))
    energies = combined.reshape(GPAD)[:G].reshape(G, 1)
    node_charges = charges_flat.reshape(N, 1)
    return (energies, node_charges)
